# Initial kernel scaffold; baseline (speedup 1.0000x reference)
import jax
import jax.numpy as jnp
from jax.experimental import pallas as pl


def kernel(token_pair_idx, token_pair_time, token_types, attr_feats_lookup, coord_feats, idx_in_lookup, n_id_lookup, W_attr, b_attr, W_coor, b_coor, basis_freq, phase, type_emb):
    raise NotImplementedError("write your pallas kernel here")


# speedup vs baseline: 3.7399x; 3.7399x over previous
import functools

import jax
import jax.numpy as jnp
from jax import lax
from jax.experimental import pallas as pl
from jax.experimental.pallas import tpu as pltpu
from jax.experimental.pallas import tpu_sc as plsc

_B, _NC, _L = 16, 16, 512
_MO, _A, _NID = 1024, 128, 64
_OUT = 352
_TOK = _B * _NC * _L
_NCORES, _NSUB = 2, 16
_NW = _NCORES * _NSUB
_TPW = _TOK // _NW
_CH = 128
_NCHUNK = _TPW // _CH


def _tw_body(a_ref, w_ref, te_ref, ba_ref, tw1_ref, tw2e_ref):
    a = a_ref[0]
    w = w_ref[...]
    tw1_ref[0] = jnp.dot(a, w[:_A, :], preferred_element_type=jnp.float32,
                         precision=lax.Precision.HIGHEST)
    y2 = jnp.dot(a, w[_A:, :], preferred_element_type=jnp.float32,
                 precision=lax.Precision.HIGHEST)
    te = te_ref[...][:, 0:_A] + ba_ref[...][None, :]
    for t in range(3):
        tw2e_ref[0, t * _MO:(t + 1) * _MO, :] = y2 + te[t][None, :]


def _mid_body(tpi_ref, tpt_ref, tty_ref, cf_ref, iil_ref, nid_ref,
              wc_ref, bc_ref, bf_ref, ph_ref, te_ref,
              mid_ref, g1_ref, g2_ref, j1_ref, j2_ref, n1e_ref, n2e_ref):
    cell = pl.program_id(0)
    b = cell // _NC
    tpt = tpt_ref[0]
    maxt = jnp.max(tpt)
    f = bf_ref[...][None, :]
    p = ph_ref[...][None, :]
    h1 = jnp.cos((maxt - tpt[:, 0:1]) * f + p)
    h2 = jnp.cos((maxt - tpt[:, 1:2]) * f + p)
    coor = jnp.dot(cf_ref[0], wc_ref[...], preferred_element_type=jnp.float32,
                   precision=lax.Precision.HIGHEST) + bc_ref[...][None, :]
    te = te_ref[...]
    tty = tty_ref[0]
    tm = te[:, 128:224]
    tsel = jnp.where(tty == 0, tm[0][None, :],
                     jnp.where(tty == 1, tm[1][None, :], tm[2][None, :]))
    mid_ref[0] = jnp.concatenate([coor, h1, h2], axis=1) + tsel
    ii = tpi_ref[0]
    g1_ref[0] = ii[:, 0:1] + b * _MO
    g2_ref[0] = ii[:, 1:2] + (b * 3) * _MO + tty * _MO
    jj = iil_ref[0]
    j1_ref[0] = jj[:, 0:1] + (cell * 3) * _NID + tty * _NID
    j2_ref[0] = jj[:, 1:2] + (cell * 3) * _NID + tty * _NID
    nid = nid_ref[0]
    tn1 = te[:, 224:288]
    tn2 = te[:, 288:352]
    n1e_ref[0] = jnp.concatenate([nid + tn1[t][None, :] for t in range(3)], axis=0)
    n2e_ref[0] = jnp.concatenate([nid + tn2[t][None, :] for t in range(3)], axis=0)


def _sc_body(tw1, tw2e, n1e, n2e, mid, g1i, g2i, j1i, j2i, out,
             i1v, i2v, k1v, k2v, g1, g2, n1, n2, midv, s1, s2, s3, s4):
    wid = lax.axis_index("s") * _NCORES + lax.axis_index("c")

    def chunk(c, carry):
        base = wid * _TPW + c * _CH
        pltpu.sync_copy(g1i.at[pl.ds(base, _CH)], i1v)
        pltpu.sync_copy(g2i.at[pl.ds(base, _CH)], i2v)
        pltpu.sync_copy(j1i.at[pl.ds(base, _CH)], k1v)
        pltpu.sync_copy(j2i.at[pl.ds(base, _CH)], k2v)
        cp1 = pltpu.async_copy(tw1.at[i1v], g1, s1)
        cp2 = pltpu.async_copy(tw2e.at[i2v], g2, s2)
        cp3 = pltpu.async_copy(n1e.at[k1v], n1, s3)
        cp4 = pltpu.async_copy(n2e.at[k2v], n2, s4)
        pltpu.sync_copy(mid.at[pl.ds(base, _CH)], midv)
        cp1.wait()
        cp2.wait()

        def addrow(r, cc):
            for j in range(8):
                sl = pl.ds(j * 16, 16)
                g1[r, sl] = g1[r, sl] + g2[r, sl]
            return cc

        lax.fori_loop(0, _CH, addrow, 0)
        cp3.wait()
        cp4.wait()
        pltpu.sync_copy(g1, out.at[pl.ds(base, _CH), pl.ds(0, 128)])
        pltpu.sync_copy(midv, out.at[pl.ds(base, _CH), pl.ds(128, 96)])
        pltpu.sync_copy(n1, out.at[pl.ds(base, _CH), pl.ds(224, 64)])
        pltpu.sync_copy(n2, out.at[pl.ds(base, _CH), pl.ds(288, 64)])
        return carry

    lax.fori_loop(0, _NCHUNK, chunk, 0)


def kernel(token_pair_idx, token_pair_time, token_types, attr_feats_lookup,
           coord_feats, idx_in_lookup, n_id_lookup,
           W_attr, b_attr, W_coor, b_coor, basis_freq, phase, type_emb):
    f32 = jnp.float32
    tw1, tw2e = pl.pallas_call(
        _tw_body,
        grid=(_B,),
        in_specs=[
            pl.BlockSpec((1, _MO, _A), lambda i: (i, 0, 0)),
            pl.BlockSpec((2 * _A, _A), lambda i: (0, 0)),
            pl.BlockSpec((3, _OUT), lambda i: (0, 0)),
            pl.BlockSpec((_A,), lambda i: (0,)),
        ],
        out_specs=[
            pl.BlockSpec((1, _MO, _A), lambda i: (i, 0, 0)),
            pl.BlockSpec((1, 3 * _MO, _A), lambda i: (i, 0, 0)),
        ],
        out_shape=[
            jax.ShapeDtypeStruct((_B, _MO, _A), f32),
            jax.ShapeDtypeStruct((_B, 3 * _MO, _A), f32),
        ],
    )(attr_feats_lookup, W_attr, type_emb, b_attr)

    ncell = _B * _NC
    tpi = token_pair_idx.reshape(ncell, _L, 2)
    tpt = token_pair_time.reshape(ncell, _L, 2)
    tty = token_types.reshape(ncell, _L, 1)
    cf = coord_feats.reshape(ncell, _L, 8)
    iil = idx_in_lookup.reshape(ncell, _L, 2)
    nid = n_id_lookup.reshape(ncell, _NID, _NID)

    mid, g1i, g2i, j1i, j2i, n1e, n2e = pl.pallas_call(
        _mid_body,
        grid=(ncell,),
        in_specs=[
            pl.BlockSpec((1, _L, 2), lambda i: (i, 0, 0)),
            pl.BlockSpec((1, _L, 2), lambda i: (i, 0, 0)),
            pl.BlockSpec((1, _L, 1), lambda i: (i, 0, 0)),
            pl.BlockSpec((1, _L, 8), lambda i: (i, 0, 0)),
            pl.BlockSpec((1, _L, 2), lambda i: (i, 0, 0)),
            pl.BlockSpec((1, _NID, _NID), lambda i: (i, 0, 0)),
            pl.BlockSpec((8, 32), lambda i: (0, 0)),
            pl.BlockSpec((32,), lambda i: (0,)),
            pl.BlockSpec((32,), lambda i: (0,)),
            pl.BlockSpec((32,), lambda i: (0,)),
            pl.BlockSpec((3, _OUT), lambda i: (0, 0)),
        ],
        out_specs=[
            pl.BlockSpec((1, _L, 96), lambda i: (i, 0, 0)),
            pl.BlockSpec((1, _L, 1), lambda i: (i, 0, 0)),
            pl.BlockSpec((1, _L, 1), lambda i: (i, 0, 0)),
            pl.BlockSpec((1, _L, 1), lambda i: (i, 0, 0)),
            pl.BlockSpec((1, _L, 1), lambda i: (i, 0, 0)),
            pl.BlockSpec((1, 3 * _NID, _NID), lambda i: (i, 0, 0)),
            pl.BlockSpec((1, 3 * _NID, _NID), lambda i: (i, 0, 0)),
        ],
        out_shape=[
            jax.ShapeDtypeStruct((ncell, _L, 96), f32),
            jax.ShapeDtypeStruct((ncell, _L, 1), jnp.int32),
            jax.ShapeDtypeStruct((ncell, _L, 1), jnp.int32),
            jax.ShapeDtypeStruct((ncell, _L, 1), jnp.int32),
            jax.ShapeDtypeStruct((ncell, _L, 1), jnp.int32),
            jax.ShapeDtypeStruct((ncell, 3 * _NID, _NID), f32),
            jax.ShapeDtypeStruct((ncell, 3 * _NID, _NID), f32),
        ],
    )(tpi, tpt, tty, cf, iil, nid, W_coor, b_coor, basis_freq, phase, type_emb)

    mesh = plsc.VectorSubcoreMesh(core_axis_name="c", subcore_axis_name="s")
    sc = functools.partial(
        pl.kernel,
        out_type=jax.ShapeDtypeStruct((_TOK, _OUT), f32),
        mesh=mesh,
        compiler_params=pltpu.CompilerParams(use_tc_tiling_on_sc=False),
        scratch_types=[
            pltpu.VMEM((_CH,), jnp.int32),
            pltpu.VMEM((_CH,), jnp.int32),
            pltpu.VMEM((_CH,), jnp.int32),
            pltpu.VMEM((_CH,), jnp.int32),
            pltpu.VMEM((_CH, _A), f32),
            pltpu.VMEM((_CH, _A), f32),
            pltpu.VMEM((_CH, _NID), f32),
            pltpu.VMEM((_CH, _NID), f32),
            pltpu.VMEM((_CH, 96), f32),
            pltpu.SemaphoreType.DMA,
            pltpu.SemaphoreType.DMA,
            pltpu.SemaphoreType.DMA,
            pltpu.SemaphoreType.DMA,
        ],
    )(_sc_body)

    out = sc(tw1.reshape(_B * _MO, _A),
             tw2e.reshape(_B * 3 * _MO, _A),
             n1e.reshape(ncell * 3 * _NID, _NID),
             n2e.reshape(ncell * 3 * _NID, _NID),
             mid.reshape(_TOK, 96),
             g1i.reshape(_TOK),
             g2i.reshape(_TOK),
             j1i.reshape(_TOK),
             j2i.reshape(_TOK))
    return out.reshape(_B, _NC, _L, _OUT)


# speedup vs baseline: 6.4518x; 1.7251x over previous
import functools

import jax
import jax.numpy as jnp
from jax import lax
from jax.experimental import pallas as pl
from jax.experimental.pallas import tpu as pltpu
from jax.experimental.pallas import tpu_sc as plsc

_B, _NC, _L = 16, 16, 512
_MO, _A, _NID = 1024, 128, 64
_OUT = 352
_TOK = _B * _NC * _L
_NCELL = _B * _NC
_NCORES, _NSUB = 2, 16
_NW = _NCORES * _NSUB
_CH = 128
_NROW = _TOK // _CH
_RPW = _NROW // _NW


def _tab_body(a_ref, w_ref, nid_ref, tw1_ref, tw2_ref, nl_ref, nr_ref):
    a = a_ref[0]
    w = w_ref[...]
    tw1_ref[0] = jnp.dot(a, w[:_A, :], preferred_element_type=jnp.float32,
                         precision=lax.Precision.HIGHEST)
    tw2_ref[0] = jnp.dot(a, w[_A:, :], preferred_element_type=jnp.float32,
                         precision=lax.Precision.HIGHEST)
    nid = nid_ref[0]
    z32 = jnp.zeros((_NC * _NID, 32), jnp.float32)
    z64 = jnp.zeros((_NC * _NID, _NID), jnp.float32)
    nl_ref[0] = jnp.concatenate([nid[:, 32:64], z64, nid[:, 0:32]], axis=1)
    nr_ref[0] = jnp.concatenate([z32, nid, z32], axis=1)


def _sc_body(tw1, tw2, nl, nr, i1h, i2h, j1h, j2h, outa, outn,
             i1v, i2v, j1v, j2v, g1, g2, h1, h2, s1, s2, s3, s4):
    wid = lax.axis_index("s") * _NCORES + lax.axis_index("c")
    row0 = wid * _RPW
    b = wid // 2

    pltpu.sync_copy(i1h.at[pl.ds(row0, _RPW)], i1v)
    pltpu.sync_copy(i2h.at[pl.ds(row0, _RPW)], i2v)
    pltpu.sync_copy(j1h.at[pl.ds(row0, _RPW)], j1v)
    pltpu.sync_copy(j2h.at[pl.ds(row0, _RPW)], j2v)

    def bias_row(r, carry):
        cell = (row0 + r) // 4
        aoff = b * _MO
        noff = cell * _NID
        for j in range(8):
            sl = pl.ds(j * 16, 16)
            i1v[r, sl] = i1v[r, sl] + aoff
            i2v[r, sl] = i2v[r, sl] + aoff
            j1v[r, sl] = j1v[r, sl] + noff
            j2v[r, sl] = j2v[r, sl] + noff
        return carry

    lax.fori_loop(0, _RPW, bias_row, 0)

    def chunk(c, carry):
        base = (row0 + c) * _CH
        cp1 = pltpu.async_copy(tw1.at[i1v.at[c]], g1, s1)
        cp2 = pltpu.async_copy(tw2.at[i2v.at[c]], g2, s2)
        cp3 = pltpu.async_copy(nl.at[j1v.at[c]], h1, s3)
        cp4 = pltpu.async_copy(nr.at[j2v.at[c]], h2, s4)
        cp1.wait()
        cp2.wait()

        def add_a(r, cc):
            for j in range(8):
                sl = pl.ds(j * 16, 16)
                g1[r, sl] = g1[r, sl] + g2[r, sl]
            return cc

        lax.fori_loop(0, _CH, add_a, 0)
        pltpu.sync_copy(g1, outa.at[pl.ds(base, _CH)])
        cp3.wait()
        cp4.wait()

        def add_n(r, cc):
            for j in range(8):
                sl = pl.ds(j * 16, 16)
                h1[r, sl] = h1[r, sl] + h2[r, sl]
            return cc

        lax.fori_loop(0, _CH, add_n, 0)
        pltpu.sync_copy(h1, outn.at[pl.ds(base, _CH)])
        return carry

    lax.fori_loop(0, _RPW, chunk, 0)


def _prep_body(wc_ref, ba_ref, bc_ref, bf_ref, ph_ref, te_ref,
               r_ref, fp_ref, s_ref):
    f32 = jnp.float32
    wrow = jnp.concatenate([
        jnp.zeros((8, 128), f32), wc_ref[...],
        jnp.zeros((8, 192), f32)], axis=1)
    brow = jnp.concatenate([
        ba_ref[...], bc_ref[...], jnp.zeros((192,), f32)])[None, :]
    r_ref[...] = jnp.concatenate([te_ref[...], wrow, brow], axis=0)
    fcat = jnp.concatenate([bf_ref[...], bf_ref[...]])[None, :]
    pcat = jnp.concatenate([ph_ref[...], ph_ref[...]])[None, :]
    fp_ref[...] = jnp.concatenate([fcat, pcat], axis=0)
    lane = lax.broadcasted_iota(jnp.int32, (2, 64), 1)
    row = lax.broadcasted_iota(jnp.int32, (2, 64), 0)
    s_ref[...] = jnp.where((lane < 32) == (row == 0), 1.0, 0.0)


def _time_body(tpt_ref, tpf_ref, fp_ref, s_ref, ht_ref):
    maxt = jnp.max(tpf_ref[0])
    tpts = jnp.dot(tpt_ref[0], s_ref[...], preferred_element_type=jnp.float32,
                   precision=lax.Precision.HIGHEST)
    ht_ref[0] = jnp.cos((maxt - tpts) * fp_ref[0:1, :] + fp_ref[1:2, :])


def _fin_body(oa_ref, on_ref, ht_ref, tty_ref, cf_ref, r_ref, out_ref):
    tty = tty_ref[0]
    oh = (tty == jnp.arange(3, dtype=jnp.int32)[None, :]).astype(jnp.float32)
    cf = cf_ref[0]
    ones = jnp.ones((_L, 1), jnp.float32)
    z = jnp.concatenate([oh, cf, ones], axis=1)
    y = jnp.dot(z, r_ref[...],
                preferred_element_type=jnp.float32)
    zl = jnp.zeros((_L, 32), jnp.float32)
    h12 = jnp.concatenate([zl, ht_ref[0], zl], axis=1)
    onr = on_ref[0]
    lane = lax.broadcasted_iota(jnp.int32, (1, 128), 1)
    add1 = jnp.where(lane >= 96, onr, h12)
    out_ref[0, :, 0:128] = y[:, 0:128] + oa_ref[0]
    out_ref[0, :, 128:256] = y[:, 128:256] + add1
    out_ref[0, :, 256:352] = y[:, 256:352] + onr[:, 0:96]


def kernel(token_pair_idx, token_pair_time, token_types, attr_feats_lookup,
           coord_feats, idx_in_lookup, n_id_lookup,
           W_attr, b_attr, W_coor, b_coor, basis_freq, phase, type_emb):
    f32 = jnp.float32
    nid3 = n_id_lookup.reshape(_B, _NC * _NID, _NID)
    tw1, tw2, nl, nr = pl.pallas_call(
        _tab_body,
        grid=(_B,),
        in_specs=[
            pl.BlockSpec((1, _MO, _A), lambda i: (i, 0, 0)),
            pl.BlockSpec((2 * _A, _A), lambda i: (0, 0)),
            pl.BlockSpec((1, _NC * _NID, _NID), lambda i: (i, 0, 0)),
        ],
        out_specs=[
            pl.BlockSpec((1, _MO, _A), lambda i: (i, 0, 0)),
            pl.BlockSpec((1, _MO, _A), lambda i: (i, 0, 0)),
            pl.BlockSpec((1, _NC * _NID, _A), lambda i: (i, 0, 0)),
            pl.BlockSpec((1, _NC * _NID, _A), lambda i: (i, 0, 0)),
        ],
        out_shape=[
            jax.ShapeDtypeStruct((_B, _MO, _A), f32),
            jax.ShapeDtypeStruct((_B, _MO, _A), f32),
            jax.ShapeDtypeStruct((_B, _NC * _NID, _A), f32),
            jax.ShapeDtypeStruct((_B, _NC * _NID, _A), f32),
        ],
    )(attr_feats_lookup, W_attr, nid3)

    i1h = token_pair_idx[..., 0].reshape(_NROW, _CH)
    i2h = token_pair_idx[..., 1].reshape(_NROW, _CH)
    j1h = idx_in_lookup[..., 0].reshape(_NROW, _CH)
    j2h = idx_in_lookup[..., 1].reshape(_NROW, _CH)

    mesh = plsc.VectorSubcoreMesh(core_axis_name="c", subcore_axis_name="s")
    sc = functools.partial(
        pl.kernel,
        out_type=[
            jax.ShapeDtypeStruct((_TOK, _A), f32),
            jax.ShapeDtypeStruct((_TOK, _A), f32),
        ],
        mesh=mesh,
        scratch_types=[
            pltpu.VMEM((_RPW, _CH), jnp.int32),
            pltpu.VMEM((_RPW, _CH), jnp.int32),
            pltpu.VMEM((_RPW, _CH), jnp.int32),
            pltpu.VMEM((_RPW, _CH), jnp.int32),
            pltpu.VMEM((_CH, _A), f32),
            pltpu.VMEM((_CH, _A), f32),
            pltpu.VMEM((_CH, _A), f32),
            pltpu.VMEM((_CH, _A), f32),
            pltpu.SemaphoreType.DMA,
            pltpu.SemaphoreType.DMA,
            pltpu.SemaphoreType.DMA,
            pltpu.SemaphoreType.DMA,
        ],
    )(_sc_body)

    outa, outn = sc(tw1.reshape(_B * _MO, _A),
                    tw2.reshape(_B * _MO, _A),
                    nl.reshape(_NCELL * _NID, _A),
                    nr.reshape(_NCELL * _NID, _A),
                    i1h, i2h, j1h, j2h)

    rmat, fp, smat = pl.pallas_call(
        _prep_body,
        in_specs=[
            pl.BlockSpec((8, 32), lambda: (0, 0)),
            pl.BlockSpec((_A,), lambda: (0,)),
            pl.BlockSpec((32,), lambda: (0,)),
            pl.BlockSpec((32,), lambda: (0,)),
            pl.BlockSpec((32,), lambda: (0,)),
            pl.BlockSpec((3, _OUT), lambda: (0, 0)),
        ],
        out_specs=[
            pl.BlockSpec((12, _OUT), lambda: (0, 0)),
            pl.BlockSpec((2, 64), lambda: (0, 0)),
            pl.BlockSpec((2, 64), lambda: (0, 0)),
        ],
        out_shape=[
            jax.ShapeDtypeStruct((12, _OUT), f32),
            jax.ShapeDtypeStruct((2, 64), f32),
            jax.ShapeDtypeStruct((2, 64), f32),
        ],
    )(W_coor, b_attr, b_coor, basis_freq, phase, type_emb)

    tpt = token_pair_time.reshape(_NCELL, _L, 2)
    tpf = token_pair_time.reshape(_NCELL, 1, _L * 2)
    tty = token_types.reshape(_NCELL, _L, 1)
    cf = coord_feats.reshape(_NCELL, _L, 8)

    ht = pl.pallas_call(
        _time_body,
        grid=(_NCELL,),
        in_specs=[
            pl.BlockSpec((1, _L, 2), lambda i: (i, 0, 0)),
            pl.BlockSpec((1, 1, _L * 2), lambda i: (i, 0, 0)),
            pl.BlockSpec((2, 64), lambda i: (0, 0)),
            pl.BlockSpec((2, 64), lambda i: (0, 0)),
        ],
        out_specs=pl.BlockSpec((1, _L, 64), lambda i: (i, 0, 0)),
        out_shape=jax.ShapeDtypeStruct((_NCELL, _L, 64), f32),
    )(tpt, tpf, fp, smat)

    out = pl.pallas_call(
        _fin_body,
        grid=(_NCELL,),
        in_specs=[
            pl.BlockSpec((1, _L, _A), lambda i: (i, 0, 0)),
            pl.BlockSpec((1, _L, _A), lambda i: (i, 0, 0)),
            pl.BlockSpec((1, _L, 64), lambda i: (i, 0, 0)),
            pl.BlockSpec((1, _L, 1), lambda i: (i, 0, 0)),
            pl.BlockSpec((1, _L, 8), lambda i: (i, 0, 0)),
            pl.BlockSpec((12, _OUT), lambda i: (0, 0)),
        ],
        out_specs=pl.BlockSpec((1, _L, _OUT), lambda i: (i, 0, 0)),
        out_shape=jax.ShapeDtypeStruct((_NCELL, _L, _OUT), f32),
    )(outa.reshape(_NCELL, _L, _A), outn.reshape(_NCELL, _L, _A),
      ht, tty, cf, rmat)

    return out.reshape(_B, _NC, _L, _OUT)
